# Initial kernel scaffold; baseline (speedup 1.0000x reference)
import jax
import jax.numpy as jnp
from jax.experimental import pallas as pl


def kernel(timeSeries):
    raise NotImplementedError("write your pallas kernel here")


# speedup vs baseline: 215.9922x; 215.9922x over previous
import functools

import jax
import jax.numpy as jnp
from jax import lax
from jax.experimental import pallas as pl
from jax.experimental.pallas import tpu as pltpu
from jax.experimental.pallas import tpu_sc as plsc

SKIP = 2
DELAY = 4
DIM = 16

NC = 2
NS = 16
NW = NC * NS

C = 2048
SLAB = 2 * C + 72
NA = C // 2 + 16


def _tde_body(n_pts, n_chunks, ts_hbm, out_hbm, slab, abuf, bbuf, obuf, sem):
    wid = lax.axis_index("s") * NC + lax.axis_index("c")
    lanes4 = jnp.arange(16, dtype=jnp.int32) * 4

    my_chunks = (n_chunks - wid + NW - 1) // NW

    def do_chunk(tl, _):
        t = wid + tl * NW
        r0 = jnp.minimum(t * C, n_pts - C)
        s0 = 2 * r0
        s0a = (s0 // 8) * 8
        delta = (s0 - s0a).astype(jnp.int32)

        pltpu.sync_copy(ts_hbm.at[pl.ds(s0a, SLAB)], slab)

        @plsc.parallel_loop(0, NA // 16, 1, unroll=4)
        def _(i):
            idx = delta + 64 * i + lanes4
            abuf[pl.ds(16 * i, 16)] = plsc.load_gather(slab, [idx])
            bbuf[pl.ds(16 * i, 16)] = plsc.load_gather(slab, [idx + 2])

        @plsc.parallel_loop(0, C // 2, 1, unroll=8)
        def _(m):
            obuf[pl.ds(32 * m, 16)] = abuf[pl.ds(m, 16)]
            obuf[pl.ds(32 * m + 16, 16)] = bbuf[pl.ds(m, 16)]

        pltpu.sync_copy(obuf, out_hbm.at[pl.ds(DIM * r0, DIM * C)])
        return _

    lax.fori_loop(0, my_chunks, do_chunk, None)


def kernel(timeSeries):
    n = timeSeries.shape[0]
    if n == 1:
        return timeSeries
    n_pts = (n - (DIM - 1) * DELAY) // SKIP
    n_chunks = (n_pts + C - 1) // C

    ts_pad = jnp.concatenate(
        [timeSeries, jnp.zeros((SLAB,), dtype=timeSeries.dtype)]
    )

    mesh = plsc.VectorSubcoreMesh(
        core_axis_name="c", subcore_axis_name="s", num_cores=NC, num_subcores=NS
    )
    out_flat = pl.kernel(
        functools.partial(_tde_body, n_pts, n_chunks),
        out_type=jax.ShapeDtypeStruct((n_pts * DIM,), jnp.float32),
        mesh=mesh,
        scratch_types=[
            pltpu.VMEM((SLAB,), jnp.float32),
            pltpu.VMEM((NA,), jnp.float32),
            pltpu.VMEM((NA,), jnp.float32),
            pltpu.VMEM((C * DIM,), jnp.float32),
            pltpu.SemaphoreType.DMA,
        ],
        compiler_params=pltpu.CompilerParams(needs_layout_passes=False),
    )(ts_pad)
    return out_flat.reshape(n_pts, DIM)


# speedup vs baseline: 217.5747x; 1.0073x over previous
import functools

import jax
import jax.numpy as jnp
from jax import lax
from jax.experimental import pallas as pl
from jax.experimental.pallas import tpu as pltpu
from jax.experimental.pallas import tpu_sc as plsc

SKIP = 2
DELAY = 4
DIM = 16

NC = 2
NS = 16
NW = NC * NS

C = 2048
SLAB = 2 * C + 64
NA = C // 2 + 16


def _tde_body(n_pts, n_chunks, ts_hbm, out_hbm, slab, abuf, bbuf, obuf, sem):
    wid = lax.axis_index("s") * NC + lax.axis_index("c")
    lanes4 = jnp.arange(16, dtype=jnp.int32) * 4

    my_chunks = (n_chunks - wid + NW - 1) // NW

    def do_chunk(tl, _):
        t = wid + tl * NW
        r0 = jnp.minimum(t * C, n_pts - C)
        s0 = 2 * r0
        s0a = (s0 // 8) * 8
        delta = (s0 - s0a).astype(jnp.int32)

        pltpu.sync_copy(ts_hbm.at[pl.ds(s0a, SLAB)], slab)

        @plsc.parallel_loop(0, NA // 16, 1, unroll=4)
        def _(i):
            idx = delta + 64 * i + lanes4
            abuf[pl.ds(16 * i, 16)] = plsc.load_gather(
                slab, [jnp.minimum(idx, SLAB - 1)]
            )
            bbuf[pl.ds(16 * i, 16)] = plsc.load_gather(
                slab, [jnp.minimum(idx + 2, SLAB - 1)]
            )

        @plsc.parallel_loop(0, C // 2, 1, unroll=8)
        def _(m):
            obuf[pl.ds(32 * m, 16)] = abuf[pl.ds(m, 16)]
            obuf[pl.ds(32 * m + 16, 16)] = bbuf[pl.ds(m, 16)]

        pltpu.sync_copy(obuf, out_hbm.at[pl.ds(DIM * r0, DIM * C)])
        return _

    lax.fori_loop(0, my_chunks, do_chunk, None)


def kernel(timeSeries):
    n = timeSeries.shape[0]
    if n == 1:
        return timeSeries
    n_pts = (n - (DIM - 1) * DELAY) // SKIP
    n_chunks = (n_pts + C - 1) // C

    mesh = plsc.VectorSubcoreMesh(
        core_axis_name="c", subcore_axis_name="s", num_cores=NC, num_subcores=NS
    )
    out_flat = pl.kernel(
        functools.partial(_tde_body, n_pts, n_chunks),
        out_type=jax.ShapeDtypeStruct((n_pts * DIM,), jnp.float32),
        mesh=mesh,
        scratch_types=[
            pltpu.VMEM((SLAB,), jnp.float32),
            pltpu.VMEM((NA,), jnp.float32),
            pltpu.VMEM((NA,), jnp.float32),
            pltpu.VMEM((C * DIM,), jnp.float32),
            pltpu.SemaphoreType.DMA,
        ],
        compiler_params=pltpu.CompilerParams(needs_layout_passes=False),
    )(timeSeries)
    return out_flat.reshape(n_pts, DIM)
